# baseline (speedup 1.0000x reference)
import jax
import jax.numpy as jnp
from jax import lax
from jax.experimental import pallas as pl
from jax.experimental.pallas import tpu as pltpu
from jax.experimental.pallas import tpu_sc as plsc

D = 64
GRP = 128
NC, NS = 2, 16
NW = NC * NS
GPB = 4
CH = GPB * GRP
NBUF = 3


SLAB = 256
NSLAB = 3906


def _pack_body(tt_hbm, o_hbm, slab0, slab1, pk0, pk1, sr0, sr1, sw0, sw1):
    wid = lax.axis_index("s") * NC + lax.axis_index("c")
    base = wid * (NSLAB // NW) + jnp.minimum(wid, NSLAB % NW)
    n_w = NSLAB // NW + jnp.where(wid < NSLAB % NW, 1, 0)

    slabs = (slab0, slab1)
    pks = (pk0, pk1)
    sr = (sr0, sr1)
    sw = (sw0, sw1)

    lanes64 = lax.iota(jnp.int32, 16) * D

    def issue_read(s, b):
        r0 = pl.multiple_of(s * SLAB, SLAB)
        pltpu.async_copy(tt_hbm.at[:, pl.ds(r0, SLAB)], slabs[b], sr[b])

    def wait_read(b):
        pltpu.make_async_copy(tt_hbm.at[:, pl.ds(0, SLAB)], slabs[b],
                              sr[b]).wait()

    def transpose(b):
        @pl.loop(0, D)
        def _d(d):
            for j in range(SLAB // 16):
                v = slabs[b][d, pl.ds(j * 16, 16)]
                plsc.store_scatter(pks[b], [lanes64 + (j * 16 * D + d)], v)

    def issue_write(s, b):
        off = pl.multiple_of(s * (SLAB * D), SLAB * D)
        pltpu.async_copy(pks[b], o_hbm.at[pl.ds(off, SLAB * D)], sw[b])

    def wait_write(b):
        pltpu.make_async_copy(pks[b], o_hbm.at[pl.ds(0, SLAB * D)],
                              sw[b]).wait()

    issue_read(base, 0)

    @pl.loop(0, n_w)
    def _slab(i):
        for b in range(2):
            @pl.when(i % 2 == b)
            def _parity():
                wait_read(b)

                @pl.when(i + 1 < n_w)
                def _prefetch():
                    issue_read(base + i + 1, 1 - b)

                @pl.when(i >= 2)
                def _free():
                    wait_write(b)

                transpose(b)
                issue_write(base + i, b)

    for b in range(2):
        wait_write(b)


def _gather_body(x_hbm, table_hbm, out_hbm, idx_v,
                 rows0, rows1, rows2, sg0, sg1, sg2, so0, so1, so2):
    wid = lax.axis_index("s") * NC + lax.axis_index("c")
    gw = x_hbm.shape[0] // NW
    n_chunks = gw // GPB
    row_base = wid * gw * GRP

    pltpu.sync_copy(x_hbm.at[pl.ds(wid * gw, gw)], idx_v)

    rows = (rows0, rows1, rows2)
    sg = (sg0, sg1, sg2)
    so = (so0, so1, so2)

    def issue_gathers(g, b):
        for j in range(GPB):
            pltpu.async_copy(
                table_hbm.at[idx_v.at[g * GPB + j]],
                rows[b].at[pl.ds(j * GRP, GRP)],
                sg[b])

    def drain_gathers(b):
        for j in range(GPB):
            pltpu.make_async_copy(
                table_hbm.at[idx_v.at[0]],
                rows[b].at[pl.ds(j * GRP, GRP)],
                sg[b]).wait()

    def issue_writeout(g, b):
        pltpu.async_copy(rows[b], out_hbm.at[pl.ds(row_base + g * CH, CH)],
                         so[b])

    def drain_writeout(b):
        pltpu.make_async_copy(rows[b], out_hbm.at[pl.ds(row_base, CH)],
                              so[b]).wait()

    def step(g, b, wait_prev=True, issue_next=True):
        drain_gathers(b)
        issue_writeout(g, b)
        if issue_next:
            bn = (b + 2) % NBUF
            if wait_prev:
                drain_writeout(bn)
            issue_gathers(g + 2, bn)

    issue_gathers(0, 0)
    issue_gathers(1, 1)
    step(0, 0, wait_prev=False)

    steady = (n_chunks - 3) // NBUF

    @pl.loop(0, steady)
    def _steady(t):
        for k in range(NBUF):
            g = 1 + t * NBUF + k
            step(g, (1 + k) % NBUF)

    for g in range(1 + steady * NBUF, n_chunks - 2):
        step(g, g % NBUF)
    for g in range(n_chunks - 2, n_chunks):
        step(g, g % NBUF, issue_next=False)

    for b in range(NBUF):
        drain_writeout(b)


def kernel(x, table):
    B, H = x.shape
    BT = B * H
    xf = x.astype(jnp.int32).reshape(BT // GRP, GRP)
    gather = pl.kernel(
        _gather_body,
        out_type=jax.ShapeDtypeStruct((BT, D), jnp.float32),
        mesh=plsc.VectorSubcoreMesh(core_axis_name="c", subcore_axis_name="s"),
        compiler_params=pltpu.CompilerParams(use_tc_tiling_on_sc=False),
        scratch_types=[
            pltpu.VMEM((BT // GRP // NW, GRP), jnp.int32),
            pltpu.VMEM((CH, D), jnp.float32),
            pltpu.VMEM((CH, D), jnp.float32),
            pltpu.VMEM((CH, D), jnp.float32),
            pltpu.SemaphoreType.DMA,
            pltpu.SemaphoreType.DMA,
            pltpu.SemaphoreType.DMA,
            pltpu.SemaphoreType.DMA,
            pltpu.SemaphoreType.DMA,
            pltpu.SemaphoreType.DMA,
        ],
    )
    V = table.shape[0]
    pack = pl.kernel(
        _pack_body,
        out_type=jax.ShapeDtypeStruct((V * D,), jnp.float32),
        mesh=plsc.VectorSubcoreMesh(core_axis_name="c", subcore_axis_name="s"),
        compiler_params=pltpu.CompilerParams(use_tc_tiling_on_sc=True,
                                             needs_layout_passes=False),
        scratch_types=[
            pltpu.VMEM((D, SLAB), jnp.float32),
            pltpu.VMEM((D, SLAB), jnp.float32),
            pltpu.VMEM((SLAB * D,), jnp.float32),
            pltpu.VMEM((SLAB * D,), jnp.float32),
            pltpu.SemaphoreType.DMA,
            pltpu.SemaphoreType.DMA,
            pltpu.SemaphoreType.DMA,
            pltpu.SemaphoreType.DMA,
        ],
    )
    tail = NSLAB * SLAB
    lin1d = jax.lax.dynamic_update_slice(
        pack(table.T), table[tail:].reshape((V - tail) * D), (tail * D,))
    table_lin = lin1d.reshape(V, D)

    out = gather(xf, table_lin)
    return out.reshape(B, H, D)


# speedup vs baseline: 1.4891x; 1.4891x over previous
import jax
import jax.numpy as jnp
from jax import lax
from jax.experimental import pallas as pl
from jax.experimental.pallas import tpu as pltpu
from jax.experimental.pallas import tpu_sc as plsc

D = 64
GRP = 128
NC, NS = 2, 16
NW = NC * NS
GPB = 4
CH = GPB * GRP
NBUF = 3


def _gather_body(x_hbm, table_hbm, out_hbm, idx_v,
                 rows0, rows1, rows2, sg0, sg1, sg2, so0, so1, so2):
    wid = lax.axis_index("s") * NC + lax.axis_index("c")
    gw = x_hbm.shape[0] // NW
    n_chunks = gw // GPB
    row_base = wid * gw * GRP

    pltpu.sync_copy(x_hbm.at[pl.ds(wid * gw, gw)], idx_v)

    rows = (rows0, rows1, rows2)
    sg = (sg0, sg1, sg2)
    so = (so0, so1, so2)

    def issue_gathers(g, b):
        for j in range(GPB):
            pltpu.async_copy(
                table_hbm.at[idx_v.at[g * GPB + j]],
                rows[b].at[pl.ds(j * GRP, GRP)],
                sg[b])

    def drain_gathers(b):
        for j in range(GPB):
            pltpu.make_async_copy(
                table_hbm.at[idx_v.at[0]],
                rows[b].at[pl.ds(j * GRP, GRP)],
                sg[b]).wait()

    def issue_writeout(g, b):
        pltpu.async_copy(rows[b], out_hbm.at[pl.ds(row_base + g * CH, CH)],
                         so[b])

    def drain_writeout(b):
        pltpu.make_async_copy(rows[b], out_hbm.at[pl.ds(row_base, CH)],
                              so[b]).wait()

    def step(g, b, wait_prev=True, issue_next=True):
        drain_gathers(b)
        issue_writeout(g, b)
        if issue_next:
            bn = (b + 2) % NBUF
            if wait_prev:
                drain_writeout(bn)
            issue_gathers(g + 2, bn)

    issue_gathers(0, 0)
    issue_gathers(1, 1)
    step(0, 0, wait_prev=False)

    steady = (n_chunks - 3) // NBUF

    @pl.loop(0, steady)
    def _steady(t):
        for k in range(NBUF):
            g = 1 + t * NBUF + k
            step(g, (1 + k) % NBUF)

    for g in range(1 + steady * NBUF, n_chunks - 2):
        step(g, g % NBUF)
    for g in range(n_chunks - 2, n_chunks):
        step(g, g % NBUF, issue_next=False)

    for b in range(NBUF):
        drain_writeout(b)


def kernel(x, table):
    B, H = x.shape
    BT = B * H
    xf = x.astype(jnp.int32).reshape(BT // GRP, GRP)
    gather = pl.kernel(
        _gather_body,
        out_type=jax.ShapeDtypeStruct((BT, D), jnp.float32),
        mesh=plsc.VectorSubcoreMesh(core_axis_name="c", subcore_axis_name="s"),
        compiler_params=pltpu.CompilerParams(use_tc_tiling_on_sc=False),
        scratch_types=[
            pltpu.VMEM((BT // GRP // NW, GRP), jnp.int32),
            pltpu.VMEM((CH, D), jnp.float32),
            pltpu.VMEM((CH, D), jnp.float32),
            pltpu.VMEM((CH, D), jnp.float32),
            pltpu.SemaphoreType.DMA,
            pltpu.SemaphoreType.DMA,
            pltpu.SemaphoreType.DMA,
            pltpu.SemaphoreType.DMA,
            pltpu.SemaphoreType.DMA,
            pltpu.SemaphoreType.DMA,
        ],
    )
    out = gather(xf, table)
    return out.reshape(B, H, D)
